# baseline (speedup 1.0000x reference)
import functools

import jax
import jax.numpy as jnp
from jax import lax
from jax.experimental import pallas as pl
from jax.experimental.pallas import tpu as pltpu
from jax.experimental.pallas import tpu_sc as plsc

T, D, H, E, K, CAP = 8192, 768, 3072, 64, 2, 128
D2 = D // 2
TB = 512
NB = T // TB
NSLOT = E * CAP
NSLOT_PAD = NSLOT + CAP
TRASH = NSLOT

NC, NS = 2, 16
NW = NC * NS
TW = T // NW
CH_B = 64
NCH_B = TW // CH_B
CH_D = 64
NCH_D = TW // CH_D


def _gate_body(x_ref, wgt_ref, bg_ref, probs_ref, route_ref, base_ref):
    b = pl.program_id(0)

    @pl.when(b == 0)
    def _():
        base_ref[...] = jnp.zeros_like(base_ref)

    x = x_ref[...]
    logits = jnp.dot(x, wgt_ref[...], preferred_element_type=jnp.float32)
    logits = logits + bg_ref[...]
    m = jnp.max(logits, axis=1, keepdims=True)
    ex = jnp.exp(logits - m)
    probs = ex / jnp.sum(ex, axis=1, keepdims=True)
    probs_ref[...] = probs

    eidx = lax.broadcasted_iota(jnp.int32, (TB, E), 1).astype(jnp.float32)
    p1 = jnp.max(probs, axis=1, keepdims=True)
    i1 = jnp.min(jnp.where(probs == p1, eidx, 1e6), axis=1, keepdims=True)
    m1 = eidx == i1
    p2 = jnp.max(jnp.where(m1, -jnp.inf, probs), axis=1, keepdims=True)
    i2 = jnp.min(jnp.where((probs == p2) & (~m1), eidx, 1e6), axis=1,
                 keepdims=True)
    m2 = eidx == i2

    onehot = m1.astype(jnp.float32) + m2.astype(jnp.float32)
    r = lax.broadcasted_iota(jnp.int32, (TB, TB), 0)
    c = lax.broadcasted_iota(jnp.int32, (TB, TB), 1)
    lt = (r > c).astype(jnp.float32)
    ranks = jnp.dot(lt, onehot, preferred_element_type=jnp.float32)
    ranks = ranks + base_ref[...]
    base_ref[...] = base_ref[...] + jnp.sum(onehot, axis=0, keepdims=True)

    rank1 = jnp.sum(jnp.where(m1, ranks, 0.0), axis=1, keepdims=True)
    rank2 = jnp.sum(jnp.where(m2, ranks, 0.0), axis=1, keepdims=True)
    v1 = rank1 < CAP
    v2 = rank2 < CAP
    s1 = jnp.where(v1, i1 * CAP + rank1, float(TRASH))
    s2 = jnp.where(v2, i2 * CAP + rank2, float(TRASH))
    sn = p1 + p2
    qm1 = jnp.where(v1, p1 / sn, 0.0)
    qm2 = jnp.where(v2, p2 / sn, 0.0)
    route_ref[...] = jnp.concatenate(
        [s1, s2, qm1, qm2, v1.astype(jnp.float32), v2.astype(jnp.float32),
         s1, s2], axis=1)


def _gate(x, wgt, bg2):
    return pl.pallas_call(
        _gate_body,
        grid=(NB,),
        in_specs=[
            pl.BlockSpec((TB, D), lambda b: (b, 0)),
            pl.BlockSpec((D, E), lambda b: (0, 0)),
            pl.BlockSpec((1, E), lambda b: (0, 0)),
        ],
        out_specs=[
            pl.BlockSpec((TB, E), lambda b: (b, 0)),
            pl.BlockSpec((TB, 8), lambda b: (b, 0)),
        ],
        out_shape=[
            jax.ShapeDtypeStruct((T, E), jnp.float32),
            jax.ShapeDtypeStruct((T, 8), jnp.float32),
        ],
        scratch_shapes=[pltpu.VMEM((1, E), jnp.float32)],
    )(x, wgt, bg2)


def _dispatch_body(x_hbm, s1_hbm, s2_hbm, xin_hbm,
                   idx1_v, idx2_v, xv0, xv1, semx0, semx1, sems0, sems1):
    wid = lax.axis_index("s") * NC + lax.axis_index("c")
    rowbase = wid * NCH_B
    pltpu.sync_copy(s1_hbm.at[pl.ds(rowbase, NCH_B)], idx1_v)
    pltpu.sync_copy(s2_hbm.at[pl.ds(rowbase, NCH_B)], idx2_v)
    xv = [xv0, xv1]
    semx = [semx0, semx1]
    sems = [sems0, sems1]

    def load(c):
        tok = wid * TW + c * CH_B
        return pltpu.async_copy(x_hbm.at[pl.ds(tok, CH_B)], xv[c % 2],
                                semx[c % 2])

    loads = [None] * NCH_B
    scats = [None] * NCH_B
    loads[0] = load(0)
    for c in range(NCH_B):
        if c + 1 < NCH_B:
            if c >= 1:
                scats[c - 1][0].wait()
                scats[c - 1][1].wait()
            loads[c + 1] = load(c + 1)
        loads[c].wait()
        f1 = pltpu.async_copy(xv[c % 2], xin_hbm.at[idx1_v.at[c]],
                              sems[c % 2])
        f2 = pltpu.async_copy(xv[c % 2], xin_hbm.at[idx2_v.at[c]],
                              sems[c % 2])
        scats[c] = (f1, f2)
    for c in (NCH_B - 2, NCH_B - 1):
        scats[c][0].wait()
        scats[c][1].wait()


@functools.cache
def _dispatch():
    return pl.kernel(
        _dispatch_body,
        out_type=jax.ShapeDtypeStruct((NSLOT_PAD, D2), jnp.float32),
        mesh=plsc.VectorSubcoreMesh(core_axis_name="c", subcore_axis_name="s",
                                    num_cores=NC, num_subcores=NS),
        scratch_types=[
            pltpu.VMEM((NCH_B, CH_B), jnp.int32),
            pltpu.VMEM((NCH_B, CH_B), jnp.int32),
            pltpu.VMEM((CH_B, D2), jnp.float32),
            pltpu.VMEM((CH_B, D2), jnp.float32),
            pltpu.SemaphoreType.DMA,
            pltpu.SemaphoreType.DMA,
            pltpu.SemaphoreType.DMA,
            pltpu.SemaphoreType.DMA,
        ],
    )


def _ffn_body(xin_ref, w1_ref, b1_ref, w2_ref, b2_ref, yout_ref):
    e = pl.program_id(0)
    pad = e == E
    xi = xin_ref[...].astype(jnp.float32)
    xh = lax.dot_general(xi, w1_ref[0], (((1,), (1,)), ((), ())),
                         preferred_element_type=jnp.float32)
    xh = xh + b1_ref[0]
    g = 0.5 * xh * (1.0 + lax.erf(xh * 0.7071067811865476))
    part = lax.dot_general(g, w2_ref[0], (((1,), (1,)), ((), ())),
                           preferred_element_type=jnp.float32)

    @pl.when(pad)
    def _():
        yout_ref[...] = jnp.zeros_like(yout_ref)

    @pl.when(~pad)
    def _():
        yout_ref[...] = (part + b2_ref[0]).astype(jnp.bfloat16)


def _ffn(xin, w1, b1, w2, b2):
    ce = lambda e: jnp.minimum(e, E - 1)
    return pl.pallas_call(
        _ffn_body,
        grid=(E + 1,),
        in_specs=[
            pl.BlockSpec((CAP, D), lambda e: (e, 0)),
            pl.BlockSpec((1, H, D), lambda e: (ce(e), 0, 0)),
            pl.BlockSpec((1, 1, H), lambda e: (ce(e), 0, 0)),
            pl.BlockSpec((1, D, H), lambda e: (ce(e), 0, 0)),
            pl.BlockSpec((1, 1, D), lambda e: (ce(e), 0, 0)),
        ],
        out_specs=pl.BlockSpec((CAP, D), lambda e: (e, 0)),
        out_shape=jax.ShapeDtypeStruct((NSLOT_PAD, D), jnp.bfloat16),
        compiler_params=pltpu.CompilerParams(
            dimension_semantics=("arbitrary",)),
    )(xin, w1, b1.reshape(E, 1, H), w2, b2.reshape(E, 1, D))


def _combine_body(yout_hbm, s1_hbm, s2_hbm, y1_hbm, y2_hbm,
                  idx1_v, idx2_v, b1v0, b1v1, b2v0, b2v1,
                  semg0, semg1, semo0, semo1):
    wid = lax.axis_index("s") * NC + lax.axis_index("c")
    rowbase = wid * NCH_D
    pltpu.sync_copy(s1_hbm.at[pl.ds(rowbase, NCH_D)], idx1_v)
    pltpu.sync_copy(s2_hbm.at[pl.ds(rowbase, NCH_D)], idx2_v)
    b1 = [b1v0, b1v1]
    b2 = [b2v0, b2v1]
    semg = [semg0, semg1]
    semo = [semo0, semo1]

    def gather(c):
        return (pltpu.async_copy(yout_hbm.at[idx1_v.at[c]], b1[c % 2],
                                 semg[c % 2]),
                pltpu.async_copy(yout_hbm.at[idx2_v.at[c]], b2[c % 2],
                                 semg[c % 2]))

    gs = [None] * NCH_D
    sts = [None] * NCH_D
    gs[0] = gather(0)
    for c in range(NCH_D):
        if c + 1 < NCH_D:
            if c >= 1:
                sts[c - 1][0].wait()
                sts[c - 1][1].wait()
            gs[c + 1] = gather(c + 1)
        gs[c][0].wait()
        gs[c][1].wait()
        tok = wid * TW + c * CH_D
        sts[c] = (
            pltpu.async_copy(b1[c % 2], y1_hbm.at[pl.ds(tok, CH_D)],
                             semo[c % 2]),
            pltpu.async_copy(b2[c % 2], y2_hbm.at[pl.ds(tok, CH_D)],
                             semo[c % 2]))
    for c in (NCH_D - 2, NCH_D - 1):
        sts[c][0].wait()
        sts[c][1].wait()


@functools.cache
def _combine():
    return pl.kernel(
        _combine_body,
        out_type=[
            jax.ShapeDtypeStruct((T, D2), jnp.float32),
            jax.ShapeDtypeStruct((T, D2), jnp.float32),
        ],
        mesh=plsc.VectorSubcoreMesh(core_axis_name="c", subcore_axis_name="s",
                                    num_cores=NC, num_subcores=NS),
        scratch_types=[
            pltpu.VMEM((NCH_D, CH_D), jnp.int32),
            pltpu.VMEM((NCH_D, CH_D), jnp.int32),
            pltpu.VMEM((CH_D, D2), jnp.float32),
            pltpu.VMEM((CH_D, D2), jnp.float32),
            pltpu.VMEM((CH_D, D2), jnp.float32),
            pltpu.VMEM((CH_D, D2), jnp.float32),
            pltpu.SemaphoreType.DMA,
            pltpu.SemaphoreType.DMA,
            pltpu.SemaphoreType.DMA,
            pltpu.SemaphoreType.DMA,
        ],
    )


def _finalize_body(y1_ref, y2_ref, route_ref, out_ref):
    q1 = route_ref[:, 2:3]
    q2 = route_ref[:, 3:4]
    out_ref[...] = (q1 * y1_ref[...].astype(jnp.float32)
                    + q2 * y2_ref[...].astype(jnp.float32))


def _finalize(y1b, y2b, route):
    return pl.pallas_call(
        _finalize_body,
        grid=(NB,),
        in_specs=[
            pl.BlockSpec((TB, D), lambda b: (b, 0)),
            pl.BlockSpec((TB, D), lambda b: (b, 0)),
            pl.BlockSpec((TB, 8), lambda b: (b, 0)),
        ],
        out_specs=pl.BlockSpec((TB, D), lambda b: (b, 0)),
        out_shape=jax.ShapeDtypeStruct((T, D), jnp.float32),
    )(y1b, y2b, route)


def kernel(x, Wg, bg, W1, b1, W2, b2):
    probs, route = _gate(x, Wg.T, bg.reshape(1, E))
    s1 = route[:, 0].astype(jnp.int32)
    s2 = route[:, 1].astype(jnp.int32)
    qm1 = route[:, 2]
    qm2 = route[:, 3]
    xp = lax.bitcast_convert_type(
        x.astype(jnp.bfloat16).reshape(T, D2, 2), jnp.float32)
    xin_p = _dispatch()(xp, s1.reshape(T // CH_B, CH_B),
                        s2.reshape(T // CH_B, CH_B))
    xin = lax.bitcast_convert_type(xin_p, jnp.bfloat16).reshape(NSLOT_PAD, D)
    yout = _ffn(xin, W1, b1, W2, b2)
    yout_p = lax.bitcast_convert_type(
        yout.reshape(NSLOT_PAD, D2, 2), jnp.float32)
    y1p, y2p = _combine()(yout_p, s1.reshape(T // CH_D, CH_D),
                          s2.reshape(T // CH_D, CH_D))
    y1b = lax.bitcast_convert_type(y1p, jnp.bfloat16).reshape(T, D)
    y2b = lax.bitcast_convert_type(y2p, jnp.bfloat16).reshape(T, D)
    out = _finalize(y1b, y2b, route)
    return out, probs


# speedup vs baseline: 1.5190x; 1.5190x over previous
import functools

import jax
import jax.numpy as jnp
from jax import lax
from jax.experimental import pallas as pl
from jax.experimental.pallas import tpu as pltpu
from jax.experimental.pallas import tpu_sc as plsc

T, D, H, E, K, CAP = 8192, 768, 3072, 64, 2, 128
TB = 512
NB = T // TB
NSLOT = E * CAP
NSLOT_PAD = NSLOT + CAP
TRASH = NSLOT
NC, NS = 2, 16
NW = NC * NS
TW = T // NW
CH_B = 64
NCH_B = TW // CH_B
CH_D = 16
NCH_D = TW // CH_D


def _gate_body(x_ref, wgt_ref, bg_ref, probs_ref, route_ref, base_ref):
    b = pl.program_id(0)

    @pl.when(b == 0)
    def _():
        base_ref[...] = jnp.zeros_like(base_ref)

    x = x_ref[...]
    logits = jnp.dot(x, wgt_ref[...], preferred_element_type=jnp.float32)
    logits = logits + bg_ref[...]
    m = jnp.max(logits, axis=1, keepdims=True)
    ex = jnp.exp(logits - m)
    probs = ex / jnp.sum(ex, axis=1, keepdims=True)
    probs_ref[...] = probs

    eidx = lax.broadcasted_iota(jnp.int32, (TB, E), 1).astype(jnp.float32)
    p1 = jnp.max(probs, axis=1, keepdims=True)
    i1 = jnp.min(jnp.where(probs == p1, eidx, 1e6), axis=1, keepdims=True)
    m1 = eidx == i1
    p2 = jnp.max(jnp.where(m1, -jnp.inf, probs), axis=1, keepdims=True)
    i2 = jnp.min(jnp.where((probs == p2) & (~m1), eidx, 1e6), axis=1,
                 keepdims=True)
    m2 = eidx == i2

    onehot = m1.astype(jnp.float32) + m2.astype(jnp.float32)
    r = lax.broadcasted_iota(jnp.int32, (TB, TB), 0)
    c = lax.broadcasted_iota(jnp.int32, (TB, TB), 1)
    lt = (r > c).astype(jnp.float32)
    ranks = jnp.dot(lt, onehot, preferred_element_type=jnp.float32)
    ranks = ranks + base_ref[...]
    base_ref[...] = base_ref[...] + jnp.sum(onehot, axis=0, keepdims=True)

    rank1 = jnp.sum(jnp.where(m1, ranks, 0.0), axis=1, keepdims=True)
    rank2 = jnp.sum(jnp.where(m2, ranks, 0.0), axis=1, keepdims=True)
    v1 = rank1 < CAP
    v2 = rank2 < CAP
    s1 = jnp.where(v1, i1 * CAP + rank1, float(TRASH))
    s2 = jnp.where(v2, i2 * CAP + rank2, float(TRASH))
    sn = p1 + p2
    qm1 = jnp.where(v1, p1 / sn, 0.0)
    qm2 = jnp.where(v2, p2 / sn, 0.0)
    route_ref[...] = jnp.concatenate(
        [s1, s2, qm1, qm2, v1.astype(jnp.float32), v2.astype(jnp.float32),
         s1, s2], axis=1)


def _gate(x, wgt, bg2):
    return pl.pallas_call(
        _gate_body,
        grid=(NB,),
        in_specs=[
            pl.BlockSpec((TB, D), lambda b: (b, 0)),
            pl.BlockSpec((D, E), lambda b: (0, 0)),
            pl.BlockSpec((1, E), lambda b: (0, 0)),
        ],
        out_specs=[
            pl.BlockSpec((TB, E), lambda b: (b, 0)),
            pl.BlockSpec((TB, 8), lambda b: (b, 0)),
        ],
        out_shape=[
            jax.ShapeDtypeStruct((T, E), jnp.float32),
            jax.ShapeDtypeStruct((T, 8), jnp.float32),
        ],
        scratch_shapes=[pltpu.VMEM((1, E), jnp.float32)],
    )(x, wgt, bg2)


def _dispatch_body(x_hbm, s1_hbm, s2_hbm, xin_hbm,
                   idx_refs, xv0, xv1, semx0, semx1, sems0, sems1):
    wid = lax.axis_index("s") * NC + lax.axis_index("c")
    for c in range(NCH_B):
        pltpu.sync_copy(s1_hbm.at[pl.ds(wid * TW + c * CH_B, CH_B)],
                        idx_refs[2 * c])
        pltpu.sync_copy(s2_hbm.at[pl.ds(wid * TW + c * CH_B, CH_B)],
                        idx_refs[2 * c + 1])
    xv = [xv0, xv1]
    semx = [semx0, semx1]
    sems = [sems0, sems1]

    def load(c):
        tok = wid * TW + c * CH_B
        return pltpu.async_copy(x_hbm.at[pl.ds(tok, CH_B)], xv[c % 2],
                                semx[c % 2])

    loads = [None] * NCH_B
    scats = [None] * NCH_B
    loads[0] = load(0)
    for c in range(NCH_B):
        if c + 1 < NCH_B:
            if c >= 1:
                scats[c - 1][0].wait()
                scats[c - 1][1].wait()
            loads[c + 1] = load(c + 1)
        loads[c].wait()
        f1 = pltpu.async_copy(
            xv[c % 2],
            xin_hbm.at[plsc.Indices(idx_refs[2 * c], ignored_value=TRASH)],
            sems[c % 2])
        f2 = pltpu.async_copy(
            xv[c % 2],
            xin_hbm.at[plsc.Indices(idx_refs[2 * c + 1], ignored_value=TRASH)],
            sems[c % 2])
        scats[c] = (f1, f2)
    for c in (NCH_B - 2, NCH_B - 1):
        scats[c][0].wait()
        scats[c][1].wait()


@functools.cache
def _dispatch():
    return pl.kernel(
        lambda x, s1, s2, xin, *rest: _dispatch_body(
            x, s1, s2, xin, list(rest[:2 * NCH_B]), *rest[2 * NCH_B:]),
        out_type=jax.ShapeDtypeStruct((NSLOT_PAD, D), jnp.float32),
        mesh=plsc.VectorSubcoreMesh(core_axis_name="c", subcore_axis_name="s",
                                    num_cores=NC, num_subcores=NS),
        scratch_types=(
            [pltpu.VMEM((CH_B,), jnp.int32)] * (2 * NCH_B) + [
                pltpu.VMEM((CH_B, D), jnp.float32),
                pltpu.VMEM((CH_B, D), jnp.float32),
                pltpu.SemaphoreType.DMA,
                pltpu.SemaphoreType.DMA,
                pltpu.SemaphoreType.DMA,
                pltpu.SemaphoreType.DMA,
            ]),
    )


def _ffn_body(xin_ref, w1_ref, b1_ref, w2_ref, b2_ref, yout_ref):
    e = pl.program_id(0)
    pad = e == E
    xi = xin_ref[...]
    xh = lax.dot_general(xi, w1_ref[0], (((1,), (1,)), ((), ())),
                         preferred_element_type=jnp.float32)
    xh = xh + b1_ref[0]
    g = 0.5 * xh * (1.0 + lax.erf(xh * 0.7071067811865476))
    part = lax.dot_general(g, w2_ref[0], (((1,), (1,)), ((), ())),
                           preferred_element_type=jnp.float32)

    @pl.when(pad)
    def _():
        yout_ref[...] = jnp.zeros_like(yout_ref)

    @pl.when(~pad)
    def _():
        yout_ref[...] = part + b2_ref[0]


def _ffn(xin, w1, b1, w2, b2):
    ce = lambda e: jnp.minimum(e, E - 1)
    return pl.pallas_call(
        _ffn_body,
        grid=(E + 1,),
        in_specs=[
            pl.BlockSpec((CAP, D), lambda e: (e, 0)),
            pl.BlockSpec((1, H, D), lambda e: (ce(e), 0, 0)),
            pl.BlockSpec((1, 1, H), lambda e: (ce(e), 0, 0)),
            pl.BlockSpec((1, D, H), lambda e: (ce(e), 0, 0)),
            pl.BlockSpec((1, 1, D), lambda e: (ce(e), 0, 0)),
        ],
        out_specs=pl.BlockSpec((CAP, D), lambda e: (e, 0)),
        out_shape=jax.ShapeDtypeStruct((NSLOT_PAD, D), jnp.float32),
        compiler_params=pltpu.CompilerParams(
            dimension_semantics=("arbitrary",)),
    )(xin, w1, b1.reshape(E, 1, H), w2, b2.reshape(E, 1, D))


def _combine_body(yout_hbm, s1_hbm, s2_hbm, q1_hbm, q2_hbm, out_hbm,
                  idx_refs, q1_v, q2_v, b1v0, b1v1, b2v0, b2v1, ov0, ov1,
                  semg0, semg1, semo0, semo1):
    wid = lax.axis_index("s") * NC + lax.axis_index("c")
    for c in range(NCH_D):
        pltpu.sync_copy(s1_hbm.at[pl.ds(wid * TW + c * CH_D, CH_D)],
                        idx_refs[2 * c])
        pltpu.sync_copy(s2_hbm.at[pl.ds(wid * TW + c * CH_D, CH_D)],
                        idx_refs[2 * c + 1])
    pltpu.sync_copy(q1_hbm.at[pl.ds(wid * TW, TW)], q1_v)
    pltpu.sync_copy(q2_hbm.at[pl.ds(wid * TW, TW)], q2_v)
    b1 = [b1v0, b1v1]
    b2 = [b2v0, b2v1]
    ov = [ov0, ov1]
    semg = [semg0, semg1]
    semo = [semo0, semo1]

    def gather(c):
        i1 = plsc.Indices(idx_refs[2 * c], ignored_value=TRASH)
        i2 = plsc.Indices(idx_refs[2 * c + 1], ignored_value=TRASH)
        return (pltpu.async_copy(yout_hbm.at[i1], b1[c % 2], semg[c % 2]),
                pltpu.async_copy(yout_hbm.at[i2], b2[c % 2], semg[c % 2]))

    gs = [None] * NCH_D
    sts = [None] * NCH_D
    gs[0] = gather(0)
    zero = jnp.zeros((16,), jnp.float32)
    for c in range(NCH_D):
        if c + 1 < NCH_D:
            gs[c + 1] = gather(c + 1)
        gs[c][0].wait()
        gs[c][1].wait()
        if c >= 2:
            sts[c - 2].wait()
        qa1 = q1_v[pl.ds(c * CH_D, 16)]
        qa2 = q2_v[pl.ds(c * CH_D, 16)]
        a1 = [qa1[j] for j in range(16)]
        a2 = [qa2[j] for j in range(16)]
        b1c, b2c, ovc = b1[c % 2], b2[c % 2], ov[c % 2]

        def lane(cc, _):
            for j in range(CH_D):
                u = b1c[j, pl.ds(cc * 16, 16)]
                v = b2c[j, pl.ds(cc * 16, 16)]
                t1 = jnp.where(a1[j] > 0.0, a1[j] * u, zero)
                t2 = jnp.where(a2[j] > 0.0, a2[j] * v, zero)
                ovc[j, pl.ds(cc * 16, 16)] = t1 + t2
            return 0

        lax.fori_loop(0, D // 16, lane, 0)
        tok = wid * TW + c * CH_D
        sts[c] = pltpu.async_copy(ovc, out_hbm.at[pl.ds(tok, CH_D)],
                                  semo[c % 2])
    sts[NCH_D - 2].wait()
    sts[NCH_D - 1].wait()


@functools.cache
def _combine():
    return pl.kernel(
        lambda yo, s1, s2, q1, q2, out, *rest: _combine_body(
            yo, s1, s2, q1, q2, out, list(rest[:2 * NCH_D]),
            *rest[2 * NCH_D:]),
        out_type=jax.ShapeDtypeStruct((T, D), jnp.float32),
        mesh=plsc.VectorSubcoreMesh(core_axis_name="c", subcore_axis_name="s",
                                    num_cores=NC, num_subcores=NS),
        scratch_types=(
            [pltpu.VMEM((CH_D,), jnp.int32)] * (2 * NCH_D) + [
                pltpu.VMEM((TW,), jnp.float32),
                pltpu.VMEM((TW,), jnp.float32),
                pltpu.VMEM((CH_D, D), jnp.float32),
                pltpu.VMEM((CH_D, D), jnp.float32),
                pltpu.VMEM((CH_D, D), jnp.float32),
                pltpu.VMEM((CH_D, D), jnp.float32),
                pltpu.VMEM((CH_D, D), jnp.float32),
                pltpu.VMEM((CH_D, D), jnp.float32),
                pltpu.SemaphoreType.DMA,
                pltpu.SemaphoreType.DMA,
                pltpu.SemaphoreType.DMA,
                pltpu.SemaphoreType.DMA,
            ]),
    )


def kernel(x, Wg, bg, W1, b1, W2, b2):
    probs, route = _gate(x, Wg.T, bg.reshape(1, E))
    s1 = route[:, 0].astype(jnp.int32)
    s2 = route[:, 1].astype(jnp.int32)
    qm1 = route[:, 2]
    qm2 = route[:, 3]
    xin = _dispatch()(x, s1, s2)
    yout = _ffn(xin, W1, b1, W2, b2)
    out = _combine()(yout, s1, s2, qm1, qm2)
    return out, probs


# speedup vs baseline: 1.9616x; 1.2913x over previous
import functools

import jax
import jax.numpy as jnp
from jax import lax
from jax.experimental import pallas as pl
from jax.experimental.pallas import tpu as pltpu
from jax.experimental.pallas import tpu_sc as plsc

T, D, H, E, K, CAP = 8192, 768, 3072, 64, 2, 128
TB = 512
NB = T // TB
NSLOT = E * CAP
NSLOT_PAD = NSLOT + CAP
TRASH = NSLOT
NC, NS = 2, 16
NW = NC * NS
TW = T // NW
CH_B = 64
NCH_B = TW // CH_B
CH_D = 16
NCH_D = TW // CH_D


def _gate_body(x_ref, wgt_ref, bg_ref, probs_ref, route_ref, base_ref):
    b = pl.program_id(0)

    @pl.when(b == 0)
    def _():
        base_ref[...] = jnp.zeros_like(base_ref)

    x = x_ref[...]
    logits = jnp.dot(x, wgt_ref[...], preferred_element_type=jnp.float32)
    logits = logits + bg_ref[...]
    m = jnp.max(logits, axis=1, keepdims=True)
    ex = jnp.exp(logits - m)
    probs = ex / jnp.sum(ex, axis=1, keepdims=True)
    probs_ref[...] = probs

    eidx = lax.broadcasted_iota(jnp.int32, (TB, E), 1).astype(jnp.float32)
    p1 = jnp.max(probs, axis=1, keepdims=True)
    i1 = jnp.min(jnp.where(probs == p1, eidx, 1e6), axis=1, keepdims=True)
    m1 = eidx == i1
    p2 = jnp.max(jnp.where(m1, -jnp.inf, probs), axis=1, keepdims=True)
    i2 = jnp.min(jnp.where((probs == p2) & (~m1), eidx, 1e6), axis=1,
                 keepdims=True)
    m2 = eidx == i2

    onehot = m1.astype(jnp.float32) + m2.astype(jnp.float32)
    r = lax.broadcasted_iota(jnp.int32, (TB, TB), 0)
    c = lax.broadcasted_iota(jnp.int32, (TB, TB), 1)
    lt = (r > c).astype(jnp.float32)
    ranks = jnp.dot(lt, onehot, preferred_element_type=jnp.float32)
    ranks = ranks + base_ref[...]
    base_ref[...] = base_ref[...] + jnp.sum(onehot, axis=0, keepdims=True)

    rank1 = jnp.sum(jnp.where(m1, ranks, 0.0), axis=1, keepdims=True)
    rank2 = jnp.sum(jnp.where(m2, ranks, 0.0), axis=1, keepdims=True)
    v1 = rank1 < CAP
    v2 = rank2 < CAP
    s1 = jnp.where(v1, i1 * CAP + rank1, float(TRASH))
    s2 = jnp.where(v2, i2 * CAP + rank2, float(TRASH))
    sn = p1 + p2
    qm1 = jnp.where(v1, p1 / sn, 0.0)
    qm2 = jnp.where(v2, p2 / sn, 0.0)
    route_ref[...] = jnp.concatenate(
        [s1, s2, qm1, qm2, v1.astype(jnp.float32), v2.astype(jnp.float32),
         s1, s2], axis=1)


def _gate(x, wgt, bg2):
    return pl.pallas_call(
        _gate_body,
        grid=(NB,),
        in_specs=[
            pl.BlockSpec((TB, D), lambda b: (b, 0)),
            pl.BlockSpec((D, E), lambda b: (0, 0)),
            pl.BlockSpec((1, E), lambda b: (0, 0)),
        ],
        out_specs=[
            pl.BlockSpec((TB, E), lambda b: (b, 0)),
            pl.BlockSpec((TB, 8), lambda b: (b, 0)),
        ],
        out_shape=[
            jax.ShapeDtypeStruct((T, E), jnp.float32),
            jax.ShapeDtypeStruct((T, 8), jnp.float32),
        ],
        scratch_shapes=[pltpu.VMEM((1, E), jnp.float32)],
    )(x, wgt, bg2)


SW = NSLOT // NW
CH_G = 64
NCH_G = SW // CH_G


def _dispatch_body(x_hbm, stt_hbm, xin_hbm,
                   idx_refs, xv0, xv1, semg0, semg1, semo0, semo1):
    wid = lax.axis_index("s") * NC + lax.axis_index("c")
    slotbase = wid * SW
    for c in range(NCH_G):
        pltpu.sync_copy(stt_hbm.at[pl.ds(slotbase + c * CH_G, CH_G)],
                        idx_refs[c])
    xv = [xv0, xv1]
    semg = [semg0, semg1]
    semo = [semo0, semo1]

    def gather(c):
        return pltpu.async_copy(x_hbm.at[idx_refs[c]], xv[c % 2],
                                semg[c % 2])

    gs = [None] * NCH_G
    sts = [None] * NCH_G
    gs[0] = gather(0)
    for c in range(NCH_G):
        if c + 1 < NCH_G:
            if c >= 1:
                sts[c - 1].wait()
            gs[c + 1] = gather(c + 1)
        gs[c].wait()
        sts[c] = pltpu.async_copy(
            xv[c % 2], xin_hbm.at[pl.ds(slotbase + c * CH_G, CH_G)],
            semo[c % 2])
    sts[NCH_G - 2].wait()
    sts[NCH_G - 1].wait()


@functools.cache
def _dispatch():
    return pl.kernel(
        lambda x, stt, xin, *rest: _dispatch_body(
            x, stt, xin, list(rest[:NCH_G]), *rest[NCH_G:]),
        out_type=jax.ShapeDtypeStruct((NSLOT_PAD, D), jnp.float32),
        mesh=plsc.VectorSubcoreMesh(core_axis_name="c", subcore_axis_name="s",
                                    num_cores=NC, num_subcores=NS),
        scratch_types=(
            [pltpu.VMEM((CH_G,), jnp.int32)] * NCH_G + [
                pltpu.VMEM((CH_G, D), jnp.float32),
                pltpu.VMEM((CH_G, D), jnp.float32),
                pltpu.SemaphoreType.DMA,
                pltpu.SemaphoreType.DMA,
                pltpu.SemaphoreType.DMA,
                pltpu.SemaphoreType.DMA,
            ]),
    )


def _ffn_body(xin_ref, w1_ref, b1_ref, w2_ref, b2_ref, yout_ref):
    e = pl.program_id(0)
    pad = e == E
    xi = xin_ref[...]
    xh = lax.dot_general(xi, w1_ref[0], (((1,), (1,)), ((), ())),
                         preferred_element_type=jnp.float32)
    xh = xh + b1_ref[0]
    g = 0.5 * xh * (1.0 + lax.erf(xh * 0.7071067811865476))
    part = lax.dot_general(g, w2_ref[0], (((1,), (1,)), ((), ())),
                           preferred_element_type=jnp.float32)

    @pl.when(pad)
    def _():
        yout_ref[...] = jnp.zeros_like(yout_ref)

    @pl.when(~pad)
    def _():
        yout_ref[...] = part + b2_ref[0]


def _ffn(xin, w1, b1, w2, b2):
    ce = lambda e: jnp.minimum(e, E - 1)
    return pl.pallas_call(
        _ffn_body,
        grid=(E + 1,),
        in_specs=[
            pl.BlockSpec((CAP, D), lambda e: (e, 0)),
            pl.BlockSpec((1, H, D), lambda e: (ce(e), 0, 0)),
            pl.BlockSpec((1, 1, H), lambda e: (ce(e), 0, 0)),
            pl.BlockSpec((1, D, H), lambda e: (ce(e), 0, 0)),
            pl.BlockSpec((1, 1, D), lambda e: (ce(e), 0, 0)),
        ],
        out_specs=pl.BlockSpec((CAP, D), lambda e: (e, 0)),
        out_shape=jax.ShapeDtypeStruct((NSLOT_PAD, D), jnp.float32),
        compiler_params=pltpu.CompilerParams(
            dimension_semantics=("arbitrary",)),
    )(xin, w1, b1.reshape(E, 1, H), w2, b2.reshape(E, 1, D))


def _combine_body(yout_hbm, s1_hbm, s2_hbm, q1_hbm, q2_hbm, out_hbm,
                  idx_refs, q1_v, q2_v, b1v0, b1v1, b2v0, b2v1, ov0, ov1,
                  semg0, semg1, semo0, semo1):
    wid = lax.axis_index("s") * NC + lax.axis_index("c")
    for c in range(NCH_D):
        pltpu.sync_copy(s1_hbm.at[pl.ds(wid * TW + c * CH_D, CH_D)],
                        idx_refs[2 * c])
        pltpu.sync_copy(s2_hbm.at[pl.ds(wid * TW + c * CH_D, CH_D)],
                        idx_refs[2 * c + 1])
    pltpu.sync_copy(q1_hbm.at[pl.ds(wid * TW, TW)], q1_v)
    pltpu.sync_copy(q2_hbm.at[pl.ds(wid * TW, TW)], q2_v)
    b1 = [b1v0, b1v1]
    b2 = [b2v0, b2v1]
    ov = [ov0, ov1]
    semg = [semg0, semg1]
    semo = [semo0, semo1]

    def gather(c):
        i1 = plsc.Indices(idx_refs[2 * c], ignored_value=TRASH)
        i2 = plsc.Indices(idx_refs[2 * c + 1], ignored_value=TRASH)
        return (pltpu.async_copy(yout_hbm.at[i1], b1[c % 2], semg[c % 2]),
                pltpu.async_copy(yout_hbm.at[i2], b2[c % 2], semg[c % 2]))

    gs = [None] * NCH_D
    sts = [None] * NCH_D
    gs[0] = gather(0)
    zero = jnp.zeros((16,), jnp.float32)
    for c in range(NCH_D):
        if c + 1 < NCH_D:
            gs[c + 1] = gather(c + 1)
        gs[c][0].wait()
        gs[c][1].wait()
        if c >= 2:
            sts[c - 2].wait()
        qa1 = q1_v[pl.ds(c * CH_D, 16)]
        qa2 = q2_v[pl.ds(c * CH_D, 16)]
        a1 = [qa1[j] for j in range(16)]
        a2 = [qa2[j] for j in range(16)]
        b1c, b2c, ovc = b1[c % 2], b2[c % 2], ov[c % 2]

        def lane(cc, _):
            for j in range(CH_D):
                u = b1c[j, pl.ds(cc * 16, 16)]
                v = b2c[j, pl.ds(cc * 16, 16)]
                t1 = jnp.where(a1[j] > 0.0, a1[j] * u, zero)
                t2 = jnp.where(a2[j] > 0.0, a2[j] * v, zero)
                ovc[j, pl.ds(cc * 16, 16)] = t1 + t2
            return 0

        lax.fori_loop(0, D // 16, lane, 0)
        tok = wid * TW + c * CH_D
        sts[c] = pltpu.async_copy(ovc, out_hbm.at[pl.ds(tok, CH_D)],
                                  semo[c % 2])
    sts[NCH_D - 2].wait()
    sts[NCH_D - 1].wait()


@functools.cache
def _combine():
    return pl.kernel(
        lambda yo, s1, s2, q1, q2, out, *rest: _combine_body(
            yo, s1, s2, q1, q2, out, list(rest[:2 * NCH_D]),
            *rest[2 * NCH_D:]),
        out_type=jax.ShapeDtypeStruct((T, D), jnp.float32),
        mesh=plsc.VectorSubcoreMesh(core_axis_name="c", subcore_axis_name="s",
                                    num_cores=NC, num_subcores=NS),
        scratch_types=(
            [pltpu.VMEM((CH_D,), jnp.int32)] * (2 * NCH_D) + [
                pltpu.VMEM((TW,), jnp.float32),
                pltpu.VMEM((TW,), jnp.float32),
                pltpu.VMEM((CH_D, D), jnp.float32),
                pltpu.VMEM((CH_D, D), jnp.float32),
                pltpu.VMEM((CH_D, D), jnp.float32),
                pltpu.VMEM((CH_D, D), jnp.float32),
                pltpu.VMEM((CH_D, D), jnp.float32),
                pltpu.VMEM((CH_D, D), jnp.float32),
                pltpu.SemaphoreType.DMA,
                pltpu.SemaphoreType.DMA,
                pltpu.SemaphoreType.DMA,
                pltpu.SemaphoreType.DMA,
            ]),
    )


def kernel(x, Wg, bg, W1, b1, W2, b2):
    probs, route = _gate(x, Wg.T, bg.reshape(1, E))
    s1 = route[:, 0].astype(jnp.int32)
    s2 = route[:, 1].astype(jnp.int32)
    qm1 = route[:, 2]
    qm2 = route[:, 3]
    stt = (jnp.zeros((NSLOT_PAD,), jnp.int32)
           .at[s1].set(jnp.arange(T, dtype=jnp.int32))
           .at[s2].set(jnp.arange(T, dtype=jnp.int32)))
    xin = _dispatch()(x, stt)
    yout = _ffn(xin, W1, b1, W2, b2)
    out = _combine()(yout, s1, s2, qm1, qm2)
    return out, probs


# speedup vs baseline: 1.9775x; 1.0081x over previous
import functools

import jax
import jax.numpy as jnp
from jax import lax
from jax.experimental import pallas as pl
from jax.experimental.pallas import tpu as pltpu
from jax.experimental.pallas import tpu_sc as plsc

T, D, H, E, K, CAP = 8192, 768, 3072, 64, 2, 128
TB = 512
NB = T // TB
NSLOT = E * CAP
NSLOT_PAD = NSLOT + CAP
TRASH = NSLOT
NC, NS = 2, 16
NW = NC * NS
TW = T // NW
CH_B = 64
NCH_B = TW // CH_B
CH_D = 32
NCH_D = TW // CH_D


def _gate_body(x_ref, wgt_ref, bg_ref, probs_ref, route_ref, base_ref):
    b = pl.program_id(0)

    @pl.when(b == 0)
    def _():
        base_ref[...] = jnp.zeros_like(base_ref)

    x = x_ref[...]
    logits = jnp.dot(x, wgt_ref[...], preferred_element_type=jnp.float32)
    logits = logits + bg_ref[...]
    m = jnp.max(logits, axis=1, keepdims=True)
    ex = jnp.exp(logits - m)
    probs = ex / jnp.sum(ex, axis=1, keepdims=True)
    probs_ref[...] = probs

    eidx = lax.broadcasted_iota(jnp.int32, (TB, E), 1).astype(jnp.float32)
    p1 = jnp.max(probs, axis=1, keepdims=True)
    i1 = jnp.min(jnp.where(probs == p1, eidx, 1e6), axis=1, keepdims=True)
    m1 = eidx == i1
    p2 = jnp.max(jnp.where(m1, -jnp.inf, probs), axis=1, keepdims=True)
    i2 = jnp.min(jnp.where((probs == p2) & (~m1), eidx, 1e6), axis=1,
                 keepdims=True)
    m2 = eidx == i2

    onehot = m1.astype(jnp.float32) + m2.astype(jnp.float32)
    r = lax.broadcasted_iota(jnp.int32, (TB, TB), 0)
    c = lax.broadcasted_iota(jnp.int32, (TB, TB), 1)
    lt = (r > c).astype(jnp.float32)
    ranks = jnp.dot(lt, onehot, preferred_element_type=jnp.float32)
    ranks = ranks + base_ref[...]
    base_ref[...] = base_ref[...] + jnp.sum(onehot, axis=0, keepdims=True)

    rank1 = jnp.sum(jnp.where(m1, ranks, 0.0), axis=1, keepdims=True)
    rank2 = jnp.sum(jnp.where(m2, ranks, 0.0), axis=1, keepdims=True)
    v1 = rank1 < CAP
    v2 = rank2 < CAP
    s1 = jnp.where(v1, i1 * CAP + rank1, float(TRASH))
    s2 = jnp.where(v2, i2 * CAP + rank2, float(TRASH))
    sn = p1 + p2
    qm1 = jnp.where(v1, p1 / sn, 0.0)
    qm2 = jnp.where(v2, p2 / sn, 0.0)
    route_ref[...] = jnp.concatenate(
        [s1, s2, qm1, qm2, v1.astype(jnp.float32), v2.astype(jnp.float32),
         s1, s2], axis=1)


def _gate(x, wgt, bg2):
    return pl.pallas_call(
        _gate_body,
        grid=(NB,),
        in_specs=[
            pl.BlockSpec((TB, D), lambda b: (b, 0)),
            pl.BlockSpec((D, E), lambda b: (0, 0)),
            pl.BlockSpec((1, E), lambda b: (0, 0)),
        ],
        out_specs=[
            pl.BlockSpec((TB, E), lambda b: (b, 0)),
            pl.BlockSpec((TB, 8), lambda b: (b, 0)),
        ],
        out_shape=[
            jax.ShapeDtypeStruct((T, E), jnp.float32),
            jax.ShapeDtypeStruct((T, 8), jnp.float32),
        ],
        scratch_shapes=[pltpu.VMEM((1, E), jnp.float32)],
    )(x, wgt, bg2)


SW = NSLOT // NW
CH_G = 64
NCH_G = SW // CH_G


def _dispatch_body(x_hbm, stt_hbm, xin_hbm,
                   idx_refs, xv0, xv1, semg0, semg1, semo0, semo1):
    wid = lax.axis_index("s") * NC + lax.axis_index("c")
    slotbase = wid * SW
    for c in range(NCH_G):
        pltpu.sync_copy(stt_hbm.at[pl.ds(slotbase + c * CH_G, CH_G)],
                        idx_refs[c])
    xv = [xv0, xv1]
    semg = [semg0, semg1]
    semo = [semo0, semo1]

    def gather(c):
        return pltpu.async_copy(x_hbm.at[idx_refs[c]], xv[c % 2],
                                semg[c % 2])

    gs = [None] * NCH_G
    sts = [None] * NCH_G
    gs[0] = gather(0)
    for c in range(NCH_G):
        if c + 1 < NCH_G:
            if c >= 1:
                sts[c - 1].wait()
            gs[c + 1] = gather(c + 1)
        gs[c].wait()
        sts[c] = pltpu.async_copy(
            xv[c % 2], xin_hbm.at[pl.ds(slotbase + c * CH_G, CH_G)],
            semo[c % 2])
    sts[NCH_G - 2].wait()
    sts[NCH_G - 1].wait()


@functools.cache
def _dispatch():
    return pl.kernel(
        lambda x, stt, xin, *rest: _dispatch_body(
            x, stt, xin, list(rest[:NCH_G]), *rest[NCH_G:]),
        out_type=jax.ShapeDtypeStruct((NSLOT_PAD, D), jnp.float32),
        mesh=plsc.VectorSubcoreMesh(core_axis_name="c", subcore_axis_name="s",
                                    num_cores=NC, num_subcores=NS),
        scratch_types=(
            [pltpu.VMEM((CH_G,), jnp.int32)] * NCH_G + [
                pltpu.VMEM((CH_G, D), jnp.float32),
                pltpu.VMEM((CH_G, D), jnp.float32),
                pltpu.SemaphoreType.DMA,
                pltpu.SemaphoreType.DMA,
                pltpu.SemaphoreType.DMA,
                pltpu.SemaphoreType.DMA,
            ]),
    )


def _ffn_body(xin_ref, w1_ref, b1_ref, w2_ref, b2_ref, yout_ref):
    e = pl.program_id(0)
    pad = e == E
    xi = xin_ref[...]
    xh = lax.dot_general(xi, w1_ref[0], (((1,), (1,)), ((), ())),
                         preferred_element_type=jnp.float32)
    xh = xh + b1_ref[0]
    g = 0.5 * xh * (1.0 + lax.erf(xh * 0.7071067811865476))
    part = lax.dot_general(g, w2_ref[0], (((1,), (1,)), ((), ())),
                           preferred_element_type=jnp.float32)

    @pl.when(pad)
    def _():
        yout_ref[...] = jnp.zeros_like(yout_ref)

    @pl.when(~pad)
    def _():
        yout_ref[...] = part + b2_ref[0]


def _ffn(xin, w1, b1, w2, b2):
    ce = lambda e: jnp.minimum(e, E - 1)
    return pl.pallas_call(
        _ffn_body,
        grid=(E + 1,),
        in_specs=[
            pl.BlockSpec((CAP, D), lambda e: (e, 0)),
            pl.BlockSpec((1, H, D), lambda e: (ce(e), 0, 0)),
            pl.BlockSpec((1, 1, H), lambda e: (ce(e), 0, 0)),
            pl.BlockSpec((1, D, H), lambda e: (ce(e), 0, 0)),
            pl.BlockSpec((1, 1, D), lambda e: (ce(e), 0, 0)),
        ],
        out_specs=pl.BlockSpec((CAP, D), lambda e: (e, 0)),
        out_shape=jax.ShapeDtypeStruct((NSLOT_PAD, D), jnp.float32),
        compiler_params=pltpu.CompilerParams(
            dimension_semantics=("arbitrary",)),
    )(xin, w1, b1.reshape(E, 1, H), w2, b2.reshape(E, 1, D))


def _combine_body(yout_hbm, s1_hbm, s2_hbm, q1_hbm, q2_hbm, out_hbm,
                  idx_refs, q1_v, q2_v, b1v0, b1v1, b2v0, b2v1, ov,
                  semg0, semg1, semo):
    wid = lax.axis_index("s") * NC + lax.axis_index("c")
    for c in range(NCH_D):
        pltpu.sync_copy(s1_hbm.at[pl.ds(wid * TW + c * CH_D, CH_D)],
                        idx_refs[2 * c])
        pltpu.sync_copy(s2_hbm.at[pl.ds(wid * TW + c * CH_D, CH_D)],
                        idx_refs[2 * c + 1])
    pltpu.sync_copy(q1_hbm.at[pl.ds(wid * TW, TW)], q1_v)
    pltpu.sync_copy(q2_hbm.at[pl.ds(wid * TW, TW)], q2_v)
    b1 = [b1v0, b1v1]
    b2 = [b2v0, b2v1]
    semg = [semg0, semg1]

    def gather(c):
        i1 = plsc.Indices(idx_refs[2 * c], ignored_value=TRASH)
        i2 = plsc.Indices(idx_refs[2 * c + 1], ignored_value=TRASH)
        return (pltpu.async_copy(yout_hbm.at[i1], b1[c % 2], semg[c % 2]),
                pltpu.async_copy(yout_hbm.at[i2], b2[c % 2], semg[c % 2]))

    gs = [None] * NCH_D
    sts = [None] * NCH_D
    gs[0] = gather(0)
    zero = jnp.zeros((16,), jnp.float32)
    for c in range(NCH_D):
        if c + 1 < NCH_D:
            gs[c + 1] = gather(c + 1)
        gs[c][0].wait()
        gs[c][1].wait()
        if c >= 1:
            sts[c - 1].wait()
        a1 = []
        a2 = []
        for g in range(CH_D // 16):
            qa1 = q1_v[pl.ds(c * CH_D + g * 16, 16)]
            qa2 = q2_v[pl.ds(c * CH_D + g * 16, 16)]
            a1 += [qa1[j] for j in range(16)]
            a2 += [qa2[j] for j in range(16)]
        b1c, b2c = b1[c % 2], b2[c % 2]

        def lane(cc, _):
            for j in range(CH_D):
                u = b1c[j, pl.ds(cc * 16, 16)]
                v = b2c[j, pl.ds(cc * 16, 16)]
                t1 = jnp.where(a1[j] > 0.0, a1[j] * u, zero)
                t2 = jnp.where(a2[j] > 0.0, a2[j] * v, zero)
                ov[j, pl.ds(cc * 16, 16)] = t1 + t2
            return 0

        lax.fori_loop(0, D // 16, lane, 0)
        tok = wid * TW + c * CH_D
        sts[c] = pltpu.async_copy(ov, out_hbm.at[pl.ds(tok, CH_D)], semo)
    sts[NCH_D - 1].wait()


@functools.cache
def _combine():
    return pl.kernel(
        lambda yo, s1, s2, q1, q2, out, *rest: _combine_body(
            yo, s1, s2, q1, q2, out, list(rest[:2 * NCH_D]),
            *rest[2 * NCH_D:]),
        out_type=jax.ShapeDtypeStruct((T, D), jnp.float32),
        mesh=plsc.VectorSubcoreMesh(core_axis_name="c", subcore_axis_name="s",
                                    num_cores=NC, num_subcores=NS),
        scratch_types=(
            [pltpu.VMEM((CH_D,), jnp.int32)] * (2 * NCH_D) + [
                pltpu.VMEM((TW,), jnp.float32),
                pltpu.VMEM((TW,), jnp.float32),
                pltpu.VMEM((CH_D, D), jnp.float32),
                pltpu.VMEM((CH_D, D), jnp.float32),
                pltpu.VMEM((CH_D, D), jnp.float32),
                pltpu.VMEM((CH_D, D), jnp.float32),
                pltpu.VMEM((CH_D, D), jnp.float32),
                pltpu.SemaphoreType.DMA,
                pltpu.SemaphoreType.DMA,
                pltpu.SemaphoreType.DMA,
            ]),
    )


def kernel(x, Wg, bg, W1, b1, W2, b2):
    probs, route = _gate(x, Wg.T, bg.reshape(1, E))
    s1 = route[:, 0].astype(jnp.int32)
    s2 = route[:, 1].astype(jnp.int32)
    qm1 = route[:, 2]
    qm2 = route[:, 3]
    stt = (jnp.zeros((NSLOT_PAD,), jnp.int32)
           .at[s1].set(jnp.arange(T, dtype=jnp.int32))
           .at[s2].set(jnp.arange(T, dtype=jnp.int32)))
    xin = _dispatch()(x, stt)
    yout = _ffn(xin, W1, b1, W2, b2)
    out = _combine()(yout, s1, s2, qm1, qm2)
    return out, probs


# speedup vs baseline: 1.9842x; 1.0034x over previous
import functools

import jax
import jax.numpy as jnp
from jax import lax
from jax.experimental import pallas as pl
from jax.experimental.pallas import tpu as pltpu
from jax.experimental.pallas import tpu_sc as plsc

T, D, H, E, K, CAP = 8192, 768, 3072, 64, 2, 128
TB = 512
NB = T // TB
NSLOT = E * CAP
NSLOT_PAD = NSLOT + CAP
TRASH = NSLOT
NC, NS = 2, 16
NW = NC * NS
TW = T // NW
CH_B = 64
NCH_B = TW // CH_B
CH_D = 32
NCH_D = TW // CH_D


def _gate_body(x_ref, wgt_ref, bg_ref, probs_ref, route_ref, base_ref):
    b = pl.program_id(0)

    @pl.when(b == 0)
    def _():
        base_ref[...] = jnp.zeros_like(base_ref)

    x = x_ref[...]
    logits = jnp.dot(x, wgt_ref[...], preferred_element_type=jnp.float32)
    logits = logits + bg_ref[...]
    m = jnp.max(logits, axis=1, keepdims=True)
    ex = jnp.exp(logits - m)
    probs = ex / jnp.sum(ex, axis=1, keepdims=True)
    probs_ref[...] = probs

    eidx = lax.broadcasted_iota(jnp.int32, (TB, E), 1).astype(jnp.float32)
    p1 = jnp.max(probs, axis=1, keepdims=True)
    i1 = jnp.min(jnp.where(probs == p1, eidx, 1e6), axis=1, keepdims=True)
    m1 = eidx == i1
    p2 = jnp.max(jnp.where(m1, -jnp.inf, probs), axis=1, keepdims=True)
    i2 = jnp.min(jnp.where((probs == p2) & (~m1), eidx, 1e6), axis=1,
                 keepdims=True)
    m2 = eidx == i2

    onehot = m1.astype(jnp.float32) + m2.astype(jnp.float32)
    r = lax.broadcasted_iota(jnp.int32, (TB, TB), 0)
    c = lax.broadcasted_iota(jnp.int32, (TB, TB), 1)
    lt = (r > c).astype(jnp.float32)
    ranks = jnp.dot(lt, onehot, preferred_element_type=jnp.float32)
    ranks = ranks + base_ref[...]
    base_ref[...] = base_ref[...] + jnp.sum(onehot, axis=0, keepdims=True)

    rank1 = jnp.sum(jnp.where(m1, ranks, 0.0), axis=1, keepdims=True)
    rank2 = jnp.sum(jnp.where(m2, ranks, 0.0), axis=1, keepdims=True)
    v1 = rank1 < CAP
    v2 = rank2 < CAP
    s1 = jnp.where(v1, i1 * CAP + rank1, float(TRASH))
    s2 = jnp.where(v2, i2 * CAP + rank2, float(TRASH))
    sn = p1 + p2
    qm1 = jnp.where(v1, p1 / sn, 0.0)
    qm2 = jnp.where(v2, p2 / sn, 0.0)
    route_ref[...] = jnp.concatenate(
        [s1, s2, qm1, qm2, v1.astype(jnp.float32), v2.astype(jnp.float32),
         s1, s2], axis=1)


def _gate(x, wgt, bg2):
    return pl.pallas_call(
        _gate_body,
        grid=(NB,),
        in_specs=[
            pl.BlockSpec((TB, D), lambda b: (b, 0)),
            pl.BlockSpec((D, E), lambda b: (0, 0)),
            pl.BlockSpec((1, E), lambda b: (0, 0)),
        ],
        out_specs=[
            pl.BlockSpec((TB, E), lambda b: (b, 0)),
            pl.BlockSpec((TB, 8), lambda b: (b, 0)),
        ],
        out_shape=[
            jax.ShapeDtypeStruct((T, E), jnp.float32),
            jax.ShapeDtypeStruct((T, 8), jnp.float32),
        ],
        scratch_shapes=[pltpu.VMEM((1, E), jnp.float32)],
    )(x, wgt, bg2)


SW = NSLOT // NW
CH_G = 64
NCH_G = SW // CH_G


def _dispatch_body(x_hbm, stt_hbm, xin_hbm,
                   idx_refs, xv0, xv1, semg0, semg1, semo0, semo1):
    wid = lax.axis_index("s") * NC + lax.axis_index("c")
    slotbase = wid * SW
    for c in range(NCH_G):
        pltpu.sync_copy(stt_hbm.at[pl.ds(slotbase + c * CH_G, CH_G)],
                        idx_refs[c])
    xv = [xv0, xv1]
    semg = [semg0, semg1]
    semo = [semo0, semo1]

    def gather(c):
        return pltpu.async_copy(x_hbm.at[idx_refs[c]], xv[c % 2],
                                semg[c % 2])

    gs = [None] * NCH_G
    sts = [None] * NCH_G
    gs[0] = gather(0)
    for c in range(NCH_G):
        if c + 1 < NCH_G:
            if c >= 1:
                sts[c - 1].wait()
            gs[c + 1] = gather(c + 1)
        gs[c].wait()
        sts[c] = pltpu.async_copy(
            xv[c % 2], xin_hbm.at[pl.ds(slotbase + c * CH_G, CH_G)],
            semo[c % 2])
    sts[NCH_G - 2].wait()
    sts[NCH_G - 1].wait()


@functools.cache
def _dispatch():
    return pl.kernel(
        lambda x, stt, xin, *rest: _dispatch_body(
            x, stt, xin, list(rest[:NCH_G]), *rest[NCH_G:]),
        out_type=jax.ShapeDtypeStruct((NSLOT_PAD, D), jnp.float32),
        mesh=plsc.VectorSubcoreMesh(core_axis_name="c", subcore_axis_name="s",
                                    num_cores=NC, num_subcores=NS),
        scratch_types=(
            [pltpu.VMEM((CH_G,), jnp.int32)] * NCH_G + [
                pltpu.VMEM((CH_G, D), jnp.float32),
                pltpu.VMEM((CH_G, D), jnp.float32),
                pltpu.SemaphoreType.DMA,
                pltpu.SemaphoreType.DMA,
                pltpu.SemaphoreType.DMA,
                pltpu.SemaphoreType.DMA,
            ]),
    )


def _ffn_body(xin_ref, w1_ref, b1_ref, w2_ref, b2_ref, yout_ref):
    e = pl.program_id(0)
    pad = e == E
    xi = xin_ref[...]
    xh = lax.dot_general(xi, w1_ref[0], (((1,), (1,)), ((), ())),
                         preferred_element_type=jnp.float32)
    xh = xh + b1_ref[0]
    g = 0.5 * xh * (1.0 + lax.erf(xh * 0.7071067811865476))
    part = lax.dot_general(g, w2_ref[0], (((1,), (1,)), ((), ())),
                           preferred_element_type=jnp.float32)

    @pl.when(pad)
    def _():
        yout_ref[...] = jnp.zeros_like(yout_ref)

    @pl.when(~pad)
    def _():
        yout_ref[...] = part + b2_ref[0]


def _ffn(xin, w1, b1, w2, b2):
    ce = lambda e: jnp.minimum(e, E - 1)
    return pl.pallas_call(
        _ffn_body,
        grid=(E + 1,),
        in_specs=[
            pl.BlockSpec((CAP, D), lambda e: (e, 0)),
            pl.BlockSpec((1, H, D), lambda e: (ce(e), 0, 0)),
            pl.BlockSpec((1, 1, H), lambda e: (ce(e), 0, 0)),
            pl.BlockSpec((1, D, H), lambda e: (ce(e), 0, 0)),
            pl.BlockSpec((1, 1, D), lambda e: (ce(e), 0, 0)),
        ],
        out_specs=pl.BlockSpec((CAP, D), lambda e: (e, 0)),
        out_shape=jax.ShapeDtypeStruct((NSLOT_PAD, D), jnp.float32),
        compiler_params=pltpu.CompilerParams(
            dimension_semantics=("arbitrary",)),
    )(xin, w1, b1.reshape(E, 1, H), w2, b2.reshape(E, 1, D))


def _combine_body(yout_hbm, sc_hbm, q1_hbm, q2_hbm, out_hbm,
                  idx_refs, q1_v, q2_v, bv0, bv1, ov,
                  semg0, semg1, semo):
    wid = lax.axis_index("s") * NC + lax.axis_index("c")
    for c in range(NCH_D):
        pltpu.sync_copy(
            sc_hbm.at[pl.ds((wid * NCH_D + c) * 2 * CH_D, 2 * CH_D)],
            idx_refs[c])
    pltpu.sync_copy(q1_hbm.at[pl.ds(wid * TW, TW)], q1_v)
    pltpu.sync_copy(q2_hbm.at[pl.ds(wid * TW, TW)], q2_v)
    bv = [bv0, bv1]
    semg = [semg0, semg1]

    def gather(c):
        ii = plsc.Indices(idx_refs[c], ignored_value=TRASH)
        return pltpu.async_copy(yout_hbm.at[ii], bv[c % 2], semg[c % 2])

    gs = [None] * NCH_D
    sts = [None] * NCH_D
    gs[0] = gather(0)
    zero = jnp.zeros((16,), jnp.float32)
    for c in range(NCH_D):
        if c + 1 < NCH_D:
            gs[c + 1] = gather(c + 1)
        gs[c].wait()
        if c >= 1:
            sts[c - 1].wait()
        a1 = []
        a2 = []
        for g in range(CH_D // 16):
            qa1 = q1_v[pl.ds(c * CH_D + g * 16, 16)]
            qa2 = q2_v[pl.ds(c * CH_D + g * 16, 16)]
            a1 += [qa1[j] for j in range(16)]
            a2 += [qa2[j] for j in range(16)]
        bc = bv[c % 2]

        def lane(cc, _):
            for j in range(CH_D):
                u = bc[j, pl.ds(cc * 16, 16)]
                v = bc[CH_D + j, pl.ds(cc * 16, 16)]
                t1 = jnp.where(a1[j] > 0.0, a1[j] * u, zero)
                t2 = jnp.where(a2[j] > 0.0, a2[j] * v, zero)
                ov[j, pl.ds(cc * 16, 16)] = t1 + t2
            return 0

        lax.fori_loop(0, D // 16, lane, 0)
        tok = wid * TW + c * CH_D
        sts[c] = pltpu.async_copy(ov, out_hbm.at[pl.ds(tok, CH_D)], semo)
    sts[NCH_D - 1].wait()


@functools.cache
def _combine():
    return pl.kernel(
        lambda yo, sc, q1, q2, out, *rest: _combine_body(
            yo, sc, q1, q2, out, list(rest[:NCH_D]), *rest[NCH_D:]),
        out_type=jax.ShapeDtypeStruct((T, D), jnp.float32),
        mesh=plsc.VectorSubcoreMesh(core_axis_name="c", subcore_axis_name="s",
                                    num_cores=NC, num_subcores=NS),
        scratch_types=(
            [pltpu.VMEM((2 * CH_D,), jnp.int32)] * NCH_D + [
                pltpu.VMEM((TW,), jnp.float32),
                pltpu.VMEM((TW,), jnp.float32),
                pltpu.VMEM((2 * CH_D, D), jnp.float32),
                pltpu.VMEM((2 * CH_D, D), jnp.float32),
                pltpu.VMEM((CH_D, D), jnp.float32),
                pltpu.SemaphoreType.DMA,
                pltpu.SemaphoreType.DMA,
                pltpu.SemaphoreType.DMA,
            ]),
    )


def kernel(x, Wg, bg, W1, b1, W2, b2):
    probs, route = _gate(x, Wg.T, bg.reshape(1, E))
    s1 = route[:, 0].astype(jnp.int32)
    s2 = route[:, 1].astype(jnp.int32)
    qm1 = route[:, 2]
    qm2 = route[:, 3]
    stt = (jnp.zeros((NSLOT_PAD,), jnp.int32)
           .at[s1].set(jnp.arange(T, dtype=jnp.int32))
           .at[s2].set(jnp.arange(T, dtype=jnp.int32)))
    xin = _dispatch()(x, stt)
    yout = _ffn(xin, W1, b1, W2, b2)
    scomb = jnp.concatenate(
        [s1.reshape(T // CH_D, CH_D), s2.reshape(T // CH_D, CH_D)],
        axis=1).reshape(2 * T)
    out = _combine()(yout, scomb, qm1, qm2)
    return out, probs


# speedup vs baseline: 1.9916x; 1.0037x over previous
import functools

import jax
import jax.numpy as jnp
from jax import lax
from jax.experimental import pallas as pl
from jax.experimental.pallas import tpu as pltpu
from jax.experimental.pallas import tpu_sc as plsc

T, D, H, E, K, CAP = 8192, 768, 3072, 64, 2, 128
TB = 512
NB = T // TB
NSLOT = E * CAP
NSLOT_PAD = NSLOT + CAP
TRASH = NSLOT
NC, NS = 2, 16
NW = NC * NS
TW = T // NW
CH_B = 64
NCH_B = TW // CH_B
CH_D = 32
NCH_D = TW // CH_D


def _gate_body(x_ref, wgt_ref, bg_ref, probs_ref, route_ref, base_ref):
    b = pl.program_id(0)

    @pl.when(b == 0)
    def _():
        base_ref[...] = jnp.zeros_like(base_ref)

    x = x_ref[...]
    logits = jnp.dot(x, wgt_ref[...], preferred_element_type=jnp.float32)
    logits = logits + bg_ref[...]
    m = jnp.max(logits, axis=1, keepdims=True)
    ex = jnp.exp(logits - m)
    probs = ex / jnp.sum(ex, axis=1, keepdims=True)
    probs_ref[...] = probs

    eidx = lax.broadcasted_iota(jnp.int32, (TB, E), 1).astype(jnp.float32)
    p1 = jnp.max(probs, axis=1, keepdims=True)
    i1 = jnp.min(jnp.where(probs == p1, eidx, 1e6), axis=1, keepdims=True)
    m1 = eidx == i1
    p2 = jnp.max(jnp.where(m1, -jnp.inf, probs), axis=1, keepdims=True)
    i2 = jnp.min(jnp.where((probs == p2) & (~m1), eidx, 1e6), axis=1,
                 keepdims=True)
    m2 = eidx == i2

    onehot = m1.astype(jnp.float32) + m2.astype(jnp.float32)
    r = lax.broadcasted_iota(jnp.int32, (TB, TB), 0)
    c = lax.broadcasted_iota(jnp.int32, (TB, TB), 1)
    lt = (r > c).astype(jnp.float32)
    ranks = jnp.dot(lt, onehot, preferred_element_type=jnp.float32)
    ranks = ranks + base_ref[...]
    base_ref[...] = base_ref[...] + jnp.sum(onehot, axis=0, keepdims=True)

    rank1 = jnp.sum(jnp.where(m1, ranks, 0.0), axis=1, keepdims=True)
    rank2 = jnp.sum(jnp.where(m2, ranks, 0.0), axis=1, keepdims=True)
    v1 = rank1 < CAP
    v2 = rank2 < CAP
    s1 = jnp.where(v1, i1 * CAP + rank1, float(TRASH))
    s2 = jnp.where(v2, i2 * CAP + rank2, float(TRASH))
    sn = p1 + p2
    qm1 = jnp.where(v1, p1 / sn, 0.0)
    qm2 = jnp.where(v2, p2 / sn, 0.0)
    route_ref[...] = jnp.concatenate(
        [s1, s2, qm1, qm2, v1.astype(jnp.float32), v2.astype(jnp.float32),
         s1, s2], axis=1)


def _gate(x, wgt, bg2):
    return pl.pallas_call(
        _gate_body,
        grid=(NB,),
        in_specs=[
            pl.BlockSpec((TB, D), lambda b: (b, 0)),
            pl.BlockSpec((D, E), lambda b: (0, 0)),
            pl.BlockSpec((1, E), lambda b: (0, 0)),
        ],
        out_specs=[
            pl.BlockSpec((TB, E), lambda b: (b, 0)),
            pl.BlockSpec((TB, 8), lambda b: (b, 0)),
        ],
        out_shape=[
            jax.ShapeDtypeStruct((T, E), jnp.float32),
            jax.ShapeDtypeStruct((T, 8), jnp.float32),
        ],
        scratch_shapes=[pltpu.VMEM((1, E), jnp.float32)],
    )(x, wgt, bg2)


SW = NSLOT // NW
CH_G = 64
NCH_G = SW // CH_G


def _dispatch_body(x_hbm, stt_hbm, xin_hbm,
                   idx_refs, xv0, xv1, semg0, semg1, semo0, semo1):
    wid = lax.axis_index("s") * NC + lax.axis_index("c")
    slotbase = wid * SW
    for c in range(NCH_G):
        pltpu.sync_copy(stt_hbm.at[pl.ds(slotbase + c * CH_G, CH_G)],
                        idx_refs[c])
    xv = [xv0, xv1]
    semg = [semg0, semg1]
    semo = [semo0, semo1]

    def gather(c):
        return pltpu.async_copy(x_hbm.at[idx_refs[c]], xv[c % 2],
                                semg[c % 2])

    gs = [None] * NCH_G
    sts = [None] * NCH_G
    gs[0] = gather(0)
    for c in range(NCH_G):
        if c + 1 < NCH_G:
            if c >= 1:
                sts[c - 1].wait()
            gs[c + 1] = gather(c + 1)
        gs[c].wait()
        sts[c] = pltpu.async_copy(
            xv[c % 2], xin_hbm.at[pl.ds(slotbase + c * CH_G, CH_G)],
            semo[c % 2])
    sts[NCH_G - 2].wait()
    sts[NCH_G - 1].wait()


@functools.cache
def _dispatch():
    return pl.kernel(
        lambda x, stt, xin, *rest: _dispatch_body(
            x, stt, xin, list(rest[:NCH_G]), *rest[NCH_G:]),
        out_type=jax.ShapeDtypeStruct((NSLOT_PAD, D), jnp.float32),
        mesh=plsc.VectorSubcoreMesh(core_axis_name="c", subcore_axis_name="s",
                                    num_cores=NC, num_subcores=NS),
        scratch_types=(
            [pltpu.VMEM((CH_G,), jnp.int32)] * NCH_G + [
                pltpu.VMEM((CH_G, D), jnp.float32),
                pltpu.VMEM((CH_G, D), jnp.float32),
                pltpu.SemaphoreType.DMA,
                pltpu.SemaphoreType.DMA,
                pltpu.SemaphoreType.DMA,
                pltpu.SemaphoreType.DMA,
            ]),
    )


def _ffn_body(xin_ref, w1_ref, b1_ref, w2_ref, b2_ref, yout_ref):
    e = pl.program_id(0)
    pad = e == E
    xi = xin_ref[...]
    xh = lax.dot_general(xi, w1_ref[0], (((1,), (1,)), ((), ())),
                         preferred_element_type=jnp.float32)
    xh = xh + b1_ref[0]
    g = 0.5 * xh * (1.0 + lax.erf(xh * 0.7071067811865476))
    part = lax.dot_general(g, w2_ref[0], (((1,), (1,)), ((), ())),
                           preferred_element_type=jnp.float32)

    @pl.when(pad)
    def _():
        yout_ref[...] = jnp.zeros_like(yout_ref)

    @pl.when(~pad)
    def _():
        yout_ref[...] = part + b2_ref[0]


def _ffn(xin, w1, b1, w2, b2):
    ce = lambda e: jnp.minimum(e, E - 1)
    return pl.pallas_call(
        _ffn_body,
        grid=(E + 1,),
        in_specs=[
            pl.BlockSpec((CAP, D), lambda e: (e, 0)),
            pl.BlockSpec((1, H, D), lambda e: (ce(e), 0, 0)),
            pl.BlockSpec((1, 1, H), lambda e: (ce(e), 0, 0)),
            pl.BlockSpec((1, D, H), lambda e: (ce(e), 0, 0)),
            pl.BlockSpec((1, 1, D), lambda e: (ce(e), 0, 0)),
        ],
        out_specs=pl.BlockSpec((CAP, D), lambda e: (e, 0)),
        out_shape=jax.ShapeDtypeStruct((NSLOT_PAD, D), jnp.float32),
        compiler_params=pltpu.CompilerParams(
            dimension_semantics=("arbitrary",)),
    )(xin, w1, b1.reshape(E, 1, H), w2, b2.reshape(E, 1, D))


def _combine_body(yout_hbm, sc_hbm, q1_hbm, q2_hbm, out_hbm,
                  idx_refs, q1_v, q2_v, bv0, bv1, ov,
                  semg0, semg1, semo):
    wid = lax.axis_index("s") * NC + lax.axis_index("c")
    for c in range(NCH_D):
        pltpu.sync_copy(
            sc_hbm.at[pl.ds((wid * NCH_D + c) * 2 * CH_D, 2 * CH_D)],
            idx_refs[c])
    pltpu.sync_copy(q1_hbm.at[pl.ds(wid * TW, TW)], q1_v)
    pltpu.sync_copy(q2_hbm.at[pl.ds(wid * TW, TW)], q2_v)
    bv = [bv0, bv1]
    semg = [semg0, semg1]

    def gather(c):
        return pltpu.async_copy(yout_hbm.at[idx_refs[c]], bv[c % 2],
                                semg[c % 2])

    gs = [None] * NCH_D
    sts = [None] * NCH_D
    gs[0] = gather(0)
    zero = jnp.zeros((16,), jnp.float32)
    for c in range(NCH_D):
        if c + 1 < NCH_D:
            gs[c + 1] = gather(c + 1)
        gs[c].wait()
        if c >= 1:
            sts[c - 1].wait()
        a1 = []
        a2 = []
        for g in range(CH_D // 16):
            qa1 = q1_v[pl.ds(c * CH_D + g * 16, 16)]
            qa2 = q2_v[pl.ds(c * CH_D + g * 16, 16)]
            a1 += [qa1[j] for j in range(16)]
            a2 += [qa2[j] for j in range(16)]
        bc = bv[c % 2]

        def lane(cc, _):
            for j in range(CH_D):
                u = bc[j, pl.ds(cc * 16, 16)]
                v = bc[CH_D + j, pl.ds(cc * 16, 16)]
                ov[j, pl.ds(cc * 16, 16)] = a1[j] * u + a2[j] * v
            return 0

        lax.fori_loop(0, D // 16, lane, 0)
        tok = wid * TW + c * CH_D
        sts[c] = pltpu.async_copy(ov, out_hbm.at[pl.ds(tok, CH_D)], semo)
    sts[NCH_D - 1].wait()


@functools.cache
def _combine():
    return pl.kernel(
        lambda yo, sc, q1, q2, out, *rest: _combine_body(
            yo, sc, q1, q2, out, list(rest[:NCH_D]), *rest[NCH_D:]),
        out_type=jax.ShapeDtypeStruct((T, D), jnp.float32),
        mesh=plsc.VectorSubcoreMesh(core_axis_name="c", subcore_axis_name="s",
                                    num_cores=NC, num_subcores=NS),
        scratch_types=(
            [pltpu.VMEM((2 * CH_D,), jnp.int32)] * NCH_D + [
                pltpu.VMEM((TW,), jnp.float32),
                pltpu.VMEM((TW,), jnp.float32),
                pltpu.VMEM((2 * CH_D, D), jnp.float32),
                pltpu.VMEM((2 * CH_D, D), jnp.float32),
                pltpu.VMEM((CH_D, D), jnp.float32),
                pltpu.SemaphoreType.DMA,
                pltpu.SemaphoreType.DMA,
                pltpu.SemaphoreType.DMA,
            ]),
    )


def kernel(x, Wg, bg, W1, b1, W2, b2):
    probs, route = _gate(x, Wg.T, bg.reshape(1, E))
    s1 = route[:, 0].astype(jnp.int32)
    s2 = route[:, 1].astype(jnp.int32)
    qm1 = route[:, 2]
    qm2 = route[:, 3]
    stt = (jnp.zeros((NSLOT_PAD,), jnp.int32)
           .at[s1].set(jnp.arange(T, dtype=jnp.int32))
           .at[s2].set(jnp.arange(T, dtype=jnp.int32)))
    xin = _dispatch()(x, stt)
    yout = _ffn(xin, W1, b1, W2, b2)
    scomb = jnp.concatenate(
        [s1.reshape(T // CH_D, CH_D), s2.reshape(T // CH_D, CH_D)],
        axis=1).reshape(2 * T)
    out = _combine()(yout, scomb, qm1, qm2)
    return out, probs


# speedup vs baseline: 3.1247x; 1.5690x over previous
import functools

import jax
import jax.numpy as jnp
from jax import lax
from jax.experimental import pallas as pl
from jax.experimental.pallas import tpu as pltpu
from jax.experimental.pallas import tpu_sc as plsc

T, D, H, E, K, CAP = 8192, 768, 3072, 64, 2, 128
TB = 512
NB = T // TB
NSLOT = E * CAP
NSLOT_PAD = NSLOT + CAP
TRASH = NSLOT
NC, NS = 2, 16
NW = NC * NS
TW = T // NW
CH_B = 64
NCH_B = TW // CH_B
CH_D = 32
NCH_D = TW // CH_D


def _gate_body(x_ref, wgt_ref, bg_ref, probs_ref, route_ref, base_ref):
    b = pl.program_id(0)

    @pl.when(b == 0)
    def _():
        base_ref[...] = jnp.zeros_like(base_ref)

    x = x_ref[...]
    logits = jnp.dot(x, wgt_ref[...], preferred_element_type=jnp.float32)
    logits = logits + bg_ref[...]
    m = jnp.max(logits, axis=1, keepdims=True)
    ex = jnp.exp(logits - m)
    probs = ex / jnp.sum(ex, axis=1, keepdims=True)
    probs_ref[...] = probs

    eidx = lax.broadcasted_iota(jnp.int32, (TB, E), 1).astype(jnp.float32)
    p1 = jnp.max(probs, axis=1, keepdims=True)
    i1 = jnp.min(jnp.where(probs == p1, eidx, 1e6), axis=1, keepdims=True)
    m1 = eidx == i1
    p2 = jnp.max(jnp.where(m1, -jnp.inf, probs), axis=1, keepdims=True)
    i2 = jnp.min(jnp.where((probs == p2) & (~m1), eidx, 1e6), axis=1,
                 keepdims=True)
    m2 = eidx == i2

    onehot = m1.astype(jnp.float32) + m2.astype(jnp.float32)
    r = lax.broadcasted_iota(jnp.int32, (TB, TB), 0)
    c = lax.broadcasted_iota(jnp.int32, (TB, TB), 1)
    lt = (r > c).astype(jnp.float32)
    ranks = jnp.dot(lt, onehot, preferred_element_type=jnp.float32)
    ranks = ranks + base_ref[...]
    base_ref[...] = base_ref[...] + jnp.sum(onehot, axis=0, keepdims=True)

    rank1 = jnp.sum(jnp.where(m1, ranks, 0.0), axis=1, keepdims=True)
    rank2 = jnp.sum(jnp.where(m2, ranks, 0.0), axis=1, keepdims=True)
    v1 = rank1 < CAP
    v2 = rank2 < CAP
    ti = lax.broadcasted_iota(jnp.int32, (TB, 1), 0)
    tmod1 = (ti % CAP).astype(jnp.float32)
    tmod2 = ((ti + CAP // 2) % CAP).astype(jnp.float32)
    s1 = jnp.where(v1, i1 * CAP + rank1, TRASH + tmod1)
    s2 = jnp.where(v2, i2 * CAP + rank2, TRASH + tmod2)
    sn = p1 + p2
    qm1 = jnp.where(v1, p1 / sn, 0.0)
    qm2 = jnp.where(v2, p2 / sn, 0.0)
    route_ref[...] = jnp.concatenate(
        [s1, s2, qm1, qm2, v1.astype(jnp.float32), v2.astype(jnp.float32),
         s1, s2], axis=1)


def _gate(x, wgt, bg2):
    return pl.pallas_call(
        _gate_body,
        grid=(NB,),
        in_specs=[
            pl.BlockSpec((TB, D), lambda b: (b, 0)),
            pl.BlockSpec((D, E), lambda b: (0, 0)),
            pl.BlockSpec((1, E), lambda b: (0, 0)),
        ],
        out_specs=[
            pl.BlockSpec((TB, E), lambda b: (b, 0)),
            pl.BlockSpec((TB, 8), lambda b: (b, 0)),
        ],
        out_shape=[
            jax.ShapeDtypeStruct((T, E), jnp.float32),
            jax.ShapeDtypeStruct((T, 8), jnp.float32),
        ],
        scratch_shapes=[pltpu.VMEM((1, E), jnp.float32)],
    )(x, wgt, bg2)


SW = NSLOT // NW
CH_G = 64
NCH_G = SW // CH_G


def _dispatch_body(x_hbm, stt_hbm, xin_hbm,
                   idx_refs, xv0, xv1, semg0, semg1, semo0, semo1):
    wid = lax.axis_index("s") * NC + lax.axis_index("c")
    slotbase = wid * SW
    for c in range(NCH_G):
        pltpu.sync_copy(stt_hbm.at[pl.ds(slotbase + c * CH_G, CH_G)],
                        idx_refs[c])
    xv = [xv0, xv1]
    semg = [semg0, semg1]
    semo = [semo0, semo1]

    def gather(c):
        return pltpu.async_copy(x_hbm.at[idx_refs[c]], xv[c % 2],
                                semg[c % 2])

    gs = [None] * NCH_G
    sts = [None] * NCH_G
    gs[0] = gather(0)
    for c in range(NCH_G):
        if c + 1 < NCH_G:
            if c >= 1:
                sts[c - 1].wait()
            gs[c + 1] = gather(c + 1)
        gs[c].wait()
        sts[c] = pltpu.async_copy(
            xv[c % 2], xin_hbm.at[pl.ds(slotbase + c * CH_G, CH_G)],
            semo[c % 2])
    sts[NCH_G - 2].wait()
    sts[NCH_G - 1].wait()


@functools.cache
def _dispatch():
    return pl.kernel(
        lambda x, stt, xin, *rest: _dispatch_body(
            x, stt, xin, list(rest[:NCH_G]), *rest[NCH_G:]),
        out_type=jax.ShapeDtypeStruct((NSLOT_PAD, D), jnp.float32),
        mesh=plsc.VectorSubcoreMesh(core_axis_name="c", subcore_axis_name="s",
                                    num_cores=NC, num_subcores=NS),
        scratch_types=(
            [pltpu.VMEM((CH_G,), jnp.int32)] * NCH_G + [
                pltpu.VMEM((CH_G, D), jnp.float32),
                pltpu.VMEM((CH_G, D), jnp.float32),
                pltpu.SemaphoreType.DMA,
                pltpu.SemaphoreType.DMA,
                pltpu.SemaphoreType.DMA,
                pltpu.SemaphoreType.DMA,
            ]),
    )


def _ffn_body(xin_ref, w1_ref, b1_ref, w2_ref, b2_ref, yout_ref):
    e = pl.program_id(0)
    pad = e == E
    xi = xin_ref[...]
    xh = lax.dot_general(xi, w1_ref[0], (((1,), (1,)), ((), ())),
                         preferred_element_type=jnp.float32)
    xh = xh + b1_ref[0]
    g = 0.5 * xh * (1.0 + lax.erf(xh * 0.7071067811865476))
    part = lax.dot_general(g, w2_ref[0], (((1,), (1,)), ((), ())),
                           preferred_element_type=jnp.float32)

    @pl.when(pad)
    def _():
        yout_ref[...] = jnp.zeros_like(yout_ref)

    @pl.when(~pad)
    def _():
        yout_ref[...] = part + b2_ref[0]


def _ffn(xin, w1, b1, w2, b2):
    ce = lambda e: jnp.minimum(e, E - 1)
    return pl.pallas_call(
        _ffn_body,
        grid=(E + 1,),
        in_specs=[
            pl.BlockSpec((CAP, D), lambda e: (e, 0)),
            pl.BlockSpec((1, H, D), lambda e: (ce(e), 0, 0)),
            pl.BlockSpec((1, 1, H), lambda e: (ce(e), 0, 0)),
            pl.BlockSpec((1, D, H), lambda e: (ce(e), 0, 0)),
            pl.BlockSpec((1, 1, D), lambda e: (ce(e), 0, 0)),
        ],
        out_specs=pl.BlockSpec((CAP, D), lambda e: (e, 0)),
        out_shape=jax.ShapeDtypeStruct((NSLOT_PAD, D), jnp.float32),
        compiler_params=pltpu.CompilerParams(
            dimension_semantics=("arbitrary",)),
    )(xin, w1, b1.reshape(E, 1, H), w2, b2.reshape(E, 1, D))


def _combine_body(yout_hbm, sc_hbm, q1_hbm, q2_hbm, out_hbm,
                  idx_refs, q1_v, q2_v, bv0, bv1, ov,
                  semg0, semg1, semo):
    wid = lax.axis_index("s") * NC + lax.axis_index("c")
    for c in range(NCH_D):
        pltpu.sync_copy(
            sc_hbm.at[pl.ds((wid * NCH_D + c) * 2 * CH_D, 2 * CH_D)],
            idx_refs[c])
    pltpu.sync_copy(q1_hbm.at[pl.ds(wid * TW, TW)], q1_v)
    pltpu.sync_copy(q2_hbm.at[pl.ds(wid * TW, TW)], q2_v)
    bv = [bv0, bv1]
    semg = [semg0, semg1]

    def gather(c):
        return pltpu.async_copy(yout_hbm.at[idx_refs[c]], bv[c % 2],
                                semg[c % 2])

    gs = [None] * NCH_D
    sts = [None] * NCH_D
    gs[0] = gather(0)
    zero = jnp.zeros((16,), jnp.float32)
    for c in range(NCH_D):
        if c + 1 < NCH_D:
            gs[c + 1] = gather(c + 1)
        gs[c].wait()
        if c >= 1:
            sts[c - 1].wait()
        a1 = []
        a2 = []
        for g in range(CH_D // 16):
            qa1 = q1_v[pl.ds(c * CH_D + g * 16, 16)]
            qa2 = q2_v[pl.ds(c * CH_D + g * 16, 16)]
            a1 += [qa1[j] for j in range(16)]
            a2 += [qa2[j] for j in range(16)]
        bc = bv[c % 2]

        def lane(cc, _):
            for j in range(CH_D):
                u = bc[j, pl.ds(cc * 16, 16)]
                v = bc[CH_D + j, pl.ds(cc * 16, 16)]
                ov[j, pl.ds(cc * 16, 16)] = a1[j] * u + a2[j] * v
            return 0

        lax.fori_loop(0, D // 16, lane, 0)
        tok = wid * TW + c * CH_D
        sts[c] = pltpu.async_copy(ov, out_hbm.at[pl.ds(tok, CH_D)], semo)
    sts[NCH_D - 1].wait()


@functools.cache
def _combine():
    return pl.kernel(
        lambda yo, sc, q1, q2, out, *rest: _combine_body(
            yo, sc, q1, q2, out, list(rest[:NCH_D]), *rest[NCH_D:]),
        out_type=jax.ShapeDtypeStruct((T, D), jnp.float32),
        mesh=plsc.VectorSubcoreMesh(core_axis_name="c", subcore_axis_name="s",
                                    num_cores=NC, num_subcores=NS),
        scratch_types=(
            [pltpu.VMEM((2 * CH_D,), jnp.int32)] * NCH_D + [
                pltpu.VMEM((TW,), jnp.float32),
                pltpu.VMEM((TW,), jnp.float32),
                pltpu.VMEM((2 * CH_D, D), jnp.float32),
                pltpu.VMEM((2 * CH_D, D), jnp.float32),
                pltpu.VMEM((CH_D, D), jnp.float32),
                pltpu.SemaphoreType.DMA,
                pltpu.SemaphoreType.DMA,
                pltpu.SemaphoreType.DMA,
            ]),
    )


def kernel(x, Wg, bg, W1, b1, W2, b2):
    probs, route = _gate(x, Wg.T, bg.reshape(1, E))
    s1 = route[:, 0].astype(jnp.int32)
    s2 = route[:, 1].astype(jnp.int32)
    qm1 = route[:, 2]
    qm2 = route[:, 3]
    stt = (jnp.zeros((NSLOT_PAD,), jnp.int32)
           .at[s1].set(jnp.arange(T, dtype=jnp.int32))
           .at[s2].set(jnp.arange(T, dtype=jnp.int32)))
    xin = _dispatch()(x, stt)
    yout = _ffn(xin, W1, b1, W2, b2)
    scomb = jnp.concatenate(
        [s1.reshape(T // CH_D, CH_D), s2.reshape(T // CH_D, CH_D)],
        axis=1).reshape(2 * T)
    out = _combine()(yout, scomb, qm1, qm2)
    return out, probs
